# baseline (device time: 32741 ns/iter reference)
import jax
import jax.numpy as jnp
from jax import lax
from jax.experimental import pallas as pl
from jax.experimental.pallas import tpu as pltpu

N_DEV = 4


def kernel(x):
    m, n = x.shape
    blk = n // N_DEV
    half = m // 2

    def body(x_ref, out_ref, fwd_r, fwd_l,
             send_sems, recv_sems, fwd_send, fwd_recv, fwd2_send, fwd2_recv):
        me = lax.axis_index("i")
        right = lax.rem(me + 1, N_DEV)
        left = lax.rem(me + N_DEV - 1, N_DEV)
        diag = lax.rem(me + 2, N_DEV)

        barrier_sem = pltpu.get_barrier_semaphore()
        for nbr in (left, right):
            pl.semaphore_signal(
                barrier_sem, inc=1,
                device_id=(nbr,), device_id_type=pl.DeviceIdType.MESH,
            )
        pl.semaphore_wait(barrier_sem, 2)

        send_r = pltpu.make_async_remote_copy(
            src_ref=x_ref.at[:, pl.ds(right * blk, blk)],
            dst_ref=out_ref.at[pl.ds(me * m, m), :],
            send_sem=send_sems.at[0], recv_sem=recv_sems.at[0],
            device_id=(right,), device_id_type=pl.DeviceIdType.MESH,
        )
        send_r.start()
        send_l = pltpu.make_async_remote_copy(
            src_ref=x_ref.at[:, pl.ds(left * blk, blk)],
            dst_ref=out_ref.at[pl.ds(me * m, m), :],
            send_sem=send_sems.at[1], recv_sem=recv_sems.at[1],
            device_id=(left,), device_id_type=pl.DeviceIdType.MESH,
        )
        send_l.start()

        half_r = pltpu.make_async_remote_copy(
            src_ref=x_ref.at[pl.ds(0, half), pl.ds(diag * blk, blk)],
            dst_ref=fwd_r,
            send_sem=fwd_send.at[0], recv_sem=fwd_recv.at[0],
            device_id=(right,), device_id_type=pl.DeviceIdType.MESH,
        )
        half_r.start()
        half_l = pltpu.make_async_remote_copy(
            src_ref=x_ref.at[pl.ds(half, half), pl.ds(diag * blk, blk)],
            dst_ref=fwd_l,
            send_sem=fwd_send.at[1], recv_sem=fwd_recv.at[1],
            device_id=(left,), device_id_type=pl.DeviceIdType.MESH,
        )
        half_l.start()

        out_ref[pl.ds(me * m, m), :] = x_ref[:, pl.ds(me * blk, blk)]

        recv_half_r = pltpu.make_async_remote_copy(
            src_ref=x_ref.at[pl.ds(0, half), pl.ds(0, blk)],
            dst_ref=fwd_r,
            send_sem=fwd_send.at[0], recv_sem=fwd_recv.at[0],
            device_id=(left,), device_id_type=pl.DeviceIdType.MESH,
        )
        recv_half_r.wait_recv()
        fwd2_r = pltpu.make_async_remote_copy(
            src_ref=fwd_r,
            dst_ref=out_ref.at[pl.ds(left * m, half), :],
            send_sem=fwd2_send.at[0], recv_sem=fwd2_recv.at[0],
            device_id=(right,), device_id_type=pl.DeviceIdType.MESH,
        )
        fwd2_r.start()

        recv_half_l = pltpu.make_async_remote_copy(
            src_ref=x_ref.at[pl.ds(0, half), pl.ds(0, blk)],
            dst_ref=fwd_l,
            send_sem=fwd_send.at[1], recv_sem=fwd_recv.at[1],
            device_id=(right,), device_id_type=pl.DeviceIdType.MESH,
        )
        recv_half_l.wait_recv()
        fwd2_l = pltpu.make_async_remote_copy(
            src_ref=fwd_l,
            dst_ref=out_ref.at[pl.ds(right * m + half, half), :],
            send_sem=fwd2_send.at[1], recv_sem=fwd2_recv.at[1],
            device_id=(left,), device_id_type=pl.DeviceIdType.MESH,
        )
        fwd2_l.start()

        for idx, src in ((0, left), (1, right)):
            recv = pltpu.make_async_remote_copy(
                src_ref=x_ref.at[:, pl.ds(0, blk)],
                dst_ref=out_ref.at[pl.ds(src * m, m), :],
                send_sem=send_sems.at[idx],
                recv_sem=recv_sems.at[idx],
                device_id=(src,), device_id_type=pl.DeviceIdType.MESH,
            )
            recv.wait_recv()
        for idx, rowoff in ((0, 0), (1, half)):
            recv = pltpu.make_async_remote_copy(
                src_ref=x_ref.at[pl.ds(0, half), pl.ds(0, blk)],
                dst_ref=out_ref.at[pl.ds(diag * m + rowoff, half), :],
                send_sem=fwd2_send.at[idx],
                recv_sem=fwd2_recv.at[idx],
                device_id=(me,), device_id_type=pl.DeviceIdType.MESH,
            )
            recv.wait_recv()

        send_r.wait_send()
        send_l.wait_send()
        half_r.wait_send()
        half_l.wait_send()
        fwd2_r.wait_send()
        fwd2_l.wait_send()

    out_shape = jax.ShapeDtypeStruct((N_DEV * m, blk), x.dtype)
    return pl.pallas_call(
        body,
        out_shape=out_shape,
        in_specs=[pl.BlockSpec(memory_space=pltpu.VMEM)],
        out_specs=pl.BlockSpec(memory_space=pltpu.VMEM),
        scratch_shapes=[
            pltpu.VMEM((half, blk), x.dtype),
            pltpu.VMEM((half, blk), x.dtype),
            pltpu.SemaphoreType.DMA((2,)),
            pltpu.SemaphoreType.DMA((2,)),
            pltpu.SemaphoreType.DMA((2,)),
            pltpu.SemaphoreType.DMA((2,)),
            pltpu.SemaphoreType.DMA((2,)),
            pltpu.SemaphoreType.DMA((2,)),
        ],
        compiler_params=pltpu.CompilerParams(collective_id=0),
    )(x)
